# baseline (speedup 1.0000x reference)
import functools

import jax
import jax.numpy as jnp
from jax import lax
from jax.experimental import pallas as pl
from jax.experimental.pallas import tpu as pltpu
from jax.experimental.pallas import tpu_sc as plsc

_N_STEPS = 128
_N1 = 64
_PROP = 0.8
_R = 512
_L = 16
_NW = 32


def _sc_body(pts_hbm, occ_hbm, tmp_hbm, rd_hbm, cam_hbm, out_hbm,
             pts_b, occ_b, tmp_b, rd_b, cam_b, ot_b, cv_b, otq_b, out_b):
    nc = 2
    wid = lax.axis_index("s") * nc + lax.axis_index("c")
    n_sc = pts_hbm.shape[0]
    per_w = n_sc // _NW
    groups = per_w // _L

    i16 = lax.iota(jnp.int32, _L)
    zeros = jnp.zeros((_L,), jnp.int32)
    f32 = jnp.float32

    pltpu.sync_copy(cam_hbm, cam_b)

    def per_group(g, carry):
        base = wid * per_w + g * _L
        pltpu.sync_copy(pts_hbm.at[pl.ds(base, _L), :], pts_b)
        pltpu.sync_copy(occ_hbm.at[pl.ds(base, _L), :], occ_b)
        pltpu.sync_copy(tmp_hbm.at[pl.ds(base, _L), :], tmp_b)
        pltpu.sync_copy(rd_hbm.at[pl.ds(base, _L), :], rd_b)

        pts_last = plsc.load_gather(pts_b, [i16, zeros + (_N_STEPS - 1)])
        cv_b[pl.ds(0, _L)] = (1.0 - _PROP) / pts_last
        cv_b[pl.ds(_L, _L)] = jnp.ones((_L,), f32)

        def per_step(k, c1):
            kk = zeros + k
            cp = cv_b[pl.ds(_L, _L)]
            occv = plsc.load_gather(occ_b, [i16, kk])
            ptsv = plsc.load_gather(pts_b, [i16, kk])
            ov = _PROP * (1.0 - cp) + cv_b[pl.ds(0, _L)] * ptsv
            ot_b[pl.ds(k * _L, _L)] = ov
            cv_b[pl.ds(_L, _L)] = cp * (1.0 - occv)
            return c1

        lax.fori_loop(0, _N_STEPS, per_step, 0, unroll=False)

        def per_query(q, c2_):
            ol = ot_b[pl.ds((_N_STEPS - 1) * _L, _L)]
            tmpv = plsc.load_gather(tmp_b, [i16, zeros + q])
            b0 = q.astype(f32) * (1.0 / _N1)
            b1 = b0 + (1.0 / _N1)
            ts = (tmpv * b0 + (1.0 - tmpv) * b1) * ol
            pos = zeros
            for step in (64, 32, 16, 8, 4, 2, 1):
                v = plsc.load_gather(ot_b, [(pos + (step - 1)) * _L + i16])
                pos = jnp.where(v <= ts, pos + step, pos)
            last_ok = (pos == _N_STEPS - 1) & (ol <= ts)
            inv = jnp.where(last_ok, pos + 1, pos)
            ii = jnp.maximum(inv - 1, 0)
            si = jnp.minimum(inv, _N_STEPS - 1)
            o_inf = jnp.where(inv == 0, -ol,
                              plsc.load_gather(ot_b, [ii * _L + i16]))
            o_sup = jnp.where(inv == _N_STEPS, 2.0 * ol,
                              plsc.load_gather(ot_b, [si * _L + i16]))
            d_inf = plsc.load_gather(pts_b, [i16, ii])
            d_sup = plsc.load_gather(pts_b, [i16, si])
            denom = o_sup - o_inf
            li = denom > 1e-6 * ol
            dist = d_inf + jnp.where(
                li, (ts - o_inf) * (d_sup - d_inf) / jnp.where(li, denom, 1.0),
                0.0)
            rdx = plsc.load_gather(rd_b, [i16, zeros])
            rdy = plsc.load_gather(rd_b, [i16, zeros + 1])
            rdz = plsc.load_gather(rd_b, [i16, zeros + 2])
            q48 = q * (3 * _L)
            otq_b[pl.ds(q48, _L)] = cam_b[0] + dist * rdx
            otq_b[pl.ds(q48 + _L, _L)] = cam_b[1] + dist * rdy
            otq_b[pl.ds(q48 + 2 * _L, _L)] = cam_b[2] + dist * rdz
            return c2_

        lax.fori_loop(0, _N1, per_query, 0, unroll=False)

        def per_row(r, c3_):
            for j in range(12):
                v = plsc.load_gather(otq_b, [256 * j + i16 * _L + r])
                out_b[pl.ds(r * 192 + j * _L, _L)] = v
            return c3_

        lax.fori_loop(0, _L, per_row, 0, unroll=False)
        pltpu.sync_copy(out_b, out_hbm.at[pl.ds(base * 192, _L * 192)])
        return carry

    lax.fori_loop(0, groups, per_group, 0, unroll=False)


def _sc_sample(pts, occ, tmp, rd, cam16, n_out):
    mesh = plsc.VectorSubcoreMesh(core_axis_name="c", subcore_axis_name="s")
    f = functools.partial(
        pl.kernel, mesh=mesh,
        compiler_params=pltpu.CompilerParams(needs_layout_passes=False),
        out_type=jax.ShapeDtypeStruct((n_out * 3 * _N1,), jnp.float32),
        scratch_types=[
            pltpu.VMEM((_L, _N_STEPS), jnp.float32),
            pltpu.VMEM((_L, _N_STEPS), jnp.float32),
            pltpu.VMEM((_L, _N1), jnp.float32),
            pltpu.VMEM((_L, 3), jnp.float32),
            pltpu.VMEM((3, _L), jnp.float32),
            pltpu.VMEM((_N_STEPS * _L,), jnp.float32),
            pltpu.VMEM((2 * _L,), jnp.float32),
            pltpu.VMEM((3 * _N1 * _L,), jnp.float32),
            pltpu.VMEM((3 * _N1 * _L,), jnp.float32),
        ],
    )(_sc_body)
    return f(pts, occ, tmp, rd, cam16)


def _body(cam_ref, pts_ref, occ_ref, rd_ref, tmp_ref, out_ref):
    pts = pts_ref[...]
    occ = occ_ref[...]
    tmp = tmp_ref[...]
    R = pts.shape[0]

    lg = jnp.log(1.0 - occ)
    uk = lax.broadcasted_iota(jnp.int32, (_N_STEPS, _N_STEPS), 0)
    um = lax.broadcasted_iota(jnp.int32, (_N_STEPS, _N_STEPS), 1)
    U = (uk < um).astype(jnp.float32)
    s = lax.dot(lg, U, precision=lax.Precision.HIGHEST,
                preferred_element_type=jnp.float32)
    cpr = jnp.exp(s)
    ptsl = pts[:, _N_STEPS - 1 : _N_STEPS]
    o = _PROP * (1.0 - cpr) + (1.0 - _PROP) * (pts / ptsl)
    o = o / o[:, _N_STEPS - 1 : _N_STEPS]

    jq = lax.broadcasted_iota(jnp.int32, (R, _N1), 1).astype(jnp.float32)
    t = (jq + 1.0) * (1.0 / _N1) - tmp * (1.0 / _N1)

    pos = jnp.zeros((R, _N1), jnp.int32)
    for step in (64, 32, 16, 8, 4, 2, 1):
        v = jnp.take_along_axis(o, pos + (step - 1), axis=1)
        pos = pos + jnp.where(v <= t, step, 0)
    inv = pos + ((pos == _N_STEPS - 1) & (t >= 1.0)).astype(jnp.int32)

    oi_idx = jnp.maximum(inv - 1, 0)
    os_idx = jnp.minimum(inv, _N_STEPS - 1)
    o_inf = jnp.where(inv == 0, -1.0, jnp.take_along_axis(o, oi_idx, axis=1))
    o_sup = jnp.where(inv >= _N_STEPS, 2.0, jnp.take_along_axis(o, os_idx, axis=1))
    d_inf = jnp.take_along_axis(pts, oi_idx, axis=1)
    d_sup = jnp.take_along_axis(pts, os_idx, axis=1)

    denom = o_sup - o_inf
    li = denom > 1e-6
    dist = d_inf + jnp.where(
        li, (t - o_inf) * (d_sup - d_inf) / jnp.where(li, denom, 1.0), 0.0
    )

    m3 = lax.broadcasted_iota(jnp.int32, (R, 3 * _N1), 1)
    c3 = m3 - (m3 // 3) * 3
    dist3 = jnp.take_along_axis(dist, m3 // 3, axis=1)
    rd = rd_ref[...]
    px = cam_ref[0, 0] + dist3 * rd[:, 0:1]
    py = cam_ref[0, 1] + dist3 * rd[:, 1:2]
    pz = cam_ref[0, 2] + dist3 * rd[:, 2:3]
    out_ref[...] = jnp.where(c3 == 0, px, jnp.where(c3 == 1, py, pz))


_SC_RAYS = 512


def _body_alias(cam_ref, pts_ref, occ_ref, rd_ref, tmp_ref, alias_ref,
                out_ref):
    del alias_ref
    _body(cam_ref, pts_ref, occ_ref, rd_ref, tmp_ref, out_ref)


def kernel(pts_intervals, occ_values, ray_directions, cam_loc, tmp):
    n_rays = pts_intervals.shape[0]
    rd = ray_directions.reshape(n_rays, 3)
    cam16 = jnp.broadcast_to(cam_loc.reshape(3, 1), (3, _L))
    s = _SC_RAYS
    out_sc = _sc_sample(pts_intervals[:s], occ_values[:s], tmp[:s], rd[:s],
                        cam16, n_rays).reshape(n_rays, 3 * _N1)
    off = s // _R
    out = pl.pallas_call(
        _body_alias,
        grid=((n_rays - s) // _R,),
        in_specs=[
            pl.BlockSpec((1, 3), lambda i: (0, 0)),
            pl.BlockSpec((_R, _N_STEPS), lambda i: (i + off, 0)),
            pl.BlockSpec((_R, _N_STEPS), lambda i: (i + off, 0)),
            pl.BlockSpec((_R, 3), lambda i: (i + off, 0)),
            pl.BlockSpec((_R, _N1), lambda i: (i + off, 0)),
            pl.BlockSpec((8, 3 * _N1), lambda i: (0, 0)),
        ],
        out_specs=pl.BlockSpec((_R, 3 * _N1), lambda i: (i + off, 0)),
        out_shape=jax.ShapeDtypeStruct((n_rays, 3 * _N1), jnp.float32),
        input_output_aliases={5: 0},
    )(cam_loc, pts_intervals, occ_values, rd, tmp, out_sc)
    return out.reshape(n_rays, _N1, 3)


def _tc_sample(pts_intervals, occ_values, rd, cam_loc, tmp):
    n_rays = pts_intervals.shape[0]
    out = pl.pallas_call(
        _body,
        grid=(n_rays // _R,),
        in_specs=[
            pl.BlockSpec((1, 3), lambda i: (0, 0)),
            pl.BlockSpec((_R, _N_STEPS), lambda i: (i, 0)),
            pl.BlockSpec((_R, _N_STEPS), lambda i: (i, 0)),
            pl.BlockSpec((_R, 3), lambda i: (i, 0)),
            pl.BlockSpec((_R, _N1), lambda i: (i, 0)),
        ],
        out_specs=pl.BlockSpec((_R, 3 * _N1), lambda i: (i, 0)),
        out_shape=jax.ShapeDtypeStruct((n_rays, 3 * _N1), jnp.float32),
    )(cam_loc, pts_intervals, occ_values, rd, tmp)
    return out


# speedup vs baseline: 1.0049x; 1.0049x over previous
import functools

import jax
import jax.numpy as jnp
from jax import lax
from jax.experimental import pallas as pl
from jax.experimental.pallas import tpu as pltpu
from jax.experimental.pallas import tpu_sc as plsc

_N_STEPS = 128
_N1 = 64
_PROP = 0.8
_R = 1024
_L = 16
_NW = 32


def _sc_body(pts_hbm, occ_hbm, tmp_hbm, rd_hbm, cam_hbm, out_hbm,
             pts_b, occ_b, tmp_b, rd_b, cam_b, ot_b, cv_b, otq_b, out_b):
    nc = 2
    wid = lax.axis_index("s") * nc + lax.axis_index("c")
    n_sc = pts_hbm.shape[0]
    per_w = n_sc // _NW
    groups = per_w // _L

    i16 = lax.iota(jnp.int32, _L)
    zeros = jnp.zeros((_L,), jnp.int32)
    f32 = jnp.float32

    pltpu.sync_copy(cam_hbm, cam_b)

    def per_group(g, carry):
        base = wid * per_w + g * _L
        pltpu.sync_copy(pts_hbm.at[pl.ds(base, _L), :], pts_b)
        pltpu.sync_copy(occ_hbm.at[pl.ds(base, _L), :], occ_b)
        pltpu.sync_copy(tmp_hbm.at[pl.ds(base, _L), :], tmp_b)
        pltpu.sync_copy(rd_hbm.at[pl.ds(base, _L), :], rd_b)

        pts_last = plsc.load_gather(pts_b, [i16, zeros + (_N_STEPS - 1)])
        cv_b[pl.ds(0, _L)] = (1.0 - _PROP) / pts_last
        cv_b[pl.ds(_L, _L)] = jnp.ones((_L,), f32)

        def per_step(k, c1):
            kk = zeros + k
            cp = cv_b[pl.ds(_L, _L)]
            occv = plsc.load_gather(occ_b, [i16, kk])
            ptsv = plsc.load_gather(pts_b, [i16, kk])
            ov = _PROP * (1.0 - cp) + cv_b[pl.ds(0, _L)] * ptsv
            ot_b[pl.ds(k * _L, _L)] = ov
            cv_b[pl.ds(_L, _L)] = cp * (1.0 - occv)
            return c1

        lax.fori_loop(0, _N_STEPS, per_step, 0, unroll=False)

        def per_query(q, c2_):
            ol = ot_b[pl.ds((_N_STEPS - 1) * _L, _L)]
            tmpv = plsc.load_gather(tmp_b, [i16, zeros + q])
            b0 = q.astype(f32) * (1.0 / _N1)
            b1 = b0 + (1.0 / _N1)
            ts = (tmpv * b0 + (1.0 - tmpv) * b1) * ol
            pos = zeros
            for step in (64, 32, 16, 8, 4, 2, 1):
                v = plsc.load_gather(ot_b, [(pos + (step - 1)) * _L + i16])
                pos = jnp.where(v <= ts, pos + step, pos)
            last_ok = (pos == _N_STEPS - 1) & (ol <= ts)
            inv = jnp.where(last_ok, pos + 1, pos)
            ii = jnp.maximum(inv - 1, 0)
            si = jnp.minimum(inv, _N_STEPS - 1)
            o_inf = jnp.where(inv == 0, -ol,
                              plsc.load_gather(ot_b, [ii * _L + i16]))
            o_sup = jnp.where(inv == _N_STEPS, 2.0 * ol,
                              plsc.load_gather(ot_b, [si * _L + i16]))
            d_inf = plsc.load_gather(pts_b, [i16, ii])
            d_sup = plsc.load_gather(pts_b, [i16, si])
            denom = o_sup - o_inf
            li = denom > 1e-6 * ol
            dist = d_inf + jnp.where(
                li, (ts - o_inf) * (d_sup - d_inf) / jnp.where(li, denom, 1.0),
                0.0)
            rdx = plsc.load_gather(rd_b, [i16, zeros])
            rdy = plsc.load_gather(rd_b, [i16, zeros + 1])
            rdz = plsc.load_gather(rd_b, [i16, zeros + 2])
            q48 = q * (3 * _L)
            otq_b[pl.ds(q48, _L)] = cam_b[0] + dist * rdx
            otq_b[pl.ds(q48 + _L, _L)] = cam_b[1] + dist * rdy
            otq_b[pl.ds(q48 + 2 * _L, _L)] = cam_b[2] + dist * rdz
            return c2_

        lax.fori_loop(0, _N1, per_query, 0, unroll=False)

        def per_row(r, c3_):
            for j in range(12):
                v = plsc.load_gather(otq_b, [256 * j + i16 * _L + r])
                out_b[pl.ds(r * 192 + j * _L, _L)] = v
            return c3_

        lax.fori_loop(0, _L, per_row, 0, unroll=False)
        pltpu.sync_copy(out_b, out_hbm.at[pl.ds(base * 192, _L * 192)])
        return carry

    lax.fori_loop(0, groups, per_group, 0, unroll=False)


def _sc_sample(pts, occ, tmp, rd, cam16, n_out):
    mesh = plsc.VectorSubcoreMesh(core_axis_name="c", subcore_axis_name="s")
    f = functools.partial(
        pl.kernel, mesh=mesh,
        compiler_params=pltpu.CompilerParams(needs_layout_passes=False),
        out_type=jax.ShapeDtypeStruct((n_out * 3 * _N1,), jnp.float32),
        scratch_types=[
            pltpu.VMEM((_L, _N_STEPS), jnp.float32),
            pltpu.VMEM((_L, _N_STEPS), jnp.float32),
            pltpu.VMEM((_L, _N1), jnp.float32),
            pltpu.VMEM((_L, 3), jnp.float32),
            pltpu.VMEM((3, _L), jnp.float32),
            pltpu.VMEM((_N_STEPS * _L,), jnp.float32),
            pltpu.VMEM((2 * _L,), jnp.float32),
            pltpu.VMEM((3 * _N1 * _L,), jnp.float32),
            pltpu.VMEM((3 * _N1 * _L,), jnp.float32),
        ],
    )(_sc_body)
    return f(pts, occ, tmp, rd, cam16)


def _body(cam_ref, pts_ref, occ_ref, rd_ref, tmp_ref, out_ref):
    pts = pts_ref[...]
    occ = occ_ref[...]
    tmp = tmp_ref[...]
    R = pts.shape[0]

    lg = jnp.log(1.0 - occ)
    uk = lax.broadcasted_iota(jnp.int32, (_N_STEPS, _N_STEPS), 0)
    um = lax.broadcasted_iota(jnp.int32, (_N_STEPS, _N_STEPS), 1)
    U = (uk < um).astype(jnp.float32)
    s = lax.dot(lg, U, precision=lax.Precision.HIGHEST,
                preferred_element_type=jnp.float32)
    cpr = jnp.exp(s)
    ptsl = pts[:, _N_STEPS - 1 : _N_STEPS]
    o = _PROP * (1.0 - cpr) + (1.0 - _PROP) * (pts / ptsl)
    o = o / o[:, _N_STEPS - 1 : _N_STEPS]

    jq = lax.broadcasted_iota(jnp.int32, (R, _N1), 1).astype(jnp.float32)
    t = (jq + 1.0) * (1.0 / _N1) - tmp * (1.0 / _N1)

    pos = jnp.zeros((R, _N1), jnp.int32)
    for step in (64, 32, 16, 8, 4, 2, 1):
        v = jnp.take_along_axis(o, pos + (step - 1), axis=1)
        pos = pos + jnp.where(v <= t, step, 0)
    inv = pos + ((pos == _N_STEPS - 1) & (t >= 1.0)).astype(jnp.int32)

    oi_idx = jnp.maximum(inv - 1, 0)
    os_idx = jnp.minimum(inv, _N_STEPS - 1)
    o_inf = jnp.where(inv == 0, -1.0, jnp.take_along_axis(o, oi_idx, axis=1))
    o_sup = jnp.where(inv >= _N_STEPS, 2.0, jnp.take_along_axis(o, os_idx, axis=1))
    d_inf = jnp.take_along_axis(pts, oi_idx, axis=1)
    d_sup = jnp.take_along_axis(pts, os_idx, axis=1)

    denom = o_sup - o_inf
    li = denom > 1e-6
    dist = d_inf + jnp.where(
        li, (t - o_inf) * (d_sup - d_inf) / jnp.where(li, denom, 1.0), 0.0
    )

    m3 = lax.broadcasted_iota(jnp.int32, (R, 3 * _N1), 1)
    c3 = m3 - (m3 // 3) * 3
    dist3 = jnp.take_along_axis(dist, m3 // 3, axis=1)
    rd = rd_ref[...]
    px = cam_ref[0, 0] + dist3 * rd[:, 0:1]
    py = cam_ref[0, 1] + dist3 * rd[:, 1:2]
    pz = cam_ref[0, 2] + dist3 * rd[:, 2:3]
    out_ref[...] = jnp.where(c3 == 0, px, jnp.where(c3 == 1, py, pz))


_SC_RAYS = 1024


def _body_alias(cam_ref, pts_ref, occ_ref, rd_ref, tmp_ref, alias_ref,
                out_ref):
    del alias_ref
    _body(cam_ref, pts_ref, occ_ref, rd_ref, tmp_ref, out_ref)


def kernel(pts_intervals, occ_values, ray_directions, cam_loc, tmp):
    n_rays = pts_intervals.shape[0]
    rd = ray_directions.reshape(n_rays, 3)
    cam16 = jnp.broadcast_to(cam_loc.reshape(3, 1), (3, _L))
    s = _SC_RAYS
    out_sc = _sc_sample(pts_intervals[:s], occ_values[:s], tmp[:s], rd[:s],
                        cam16, n_rays).reshape(n_rays, 3 * _N1)
    off = s // _R
    out = pl.pallas_call(
        _body_alias,
        grid=((n_rays - s) // _R,),
        in_specs=[
            pl.BlockSpec((1, 3), lambda i: (0, 0)),
            pl.BlockSpec((_R, _N_STEPS), lambda i: (i + off, 0)),
            pl.BlockSpec((_R, _N_STEPS), lambda i: (i + off, 0)),
            pl.BlockSpec((_R, 3), lambda i: (i + off, 0)),
            pl.BlockSpec((_R, _N1), lambda i: (i + off, 0)),
            pl.BlockSpec((8, 3 * _N1), lambda i: (0, 0)),
        ],
        out_specs=pl.BlockSpec((_R, 3 * _N1), lambda i: (i + off, 0)),
        out_shape=jax.ShapeDtypeStruct((n_rays, 3 * _N1), jnp.float32),
        input_output_aliases={5: 0},
    )(cam_loc, pts_intervals, occ_values, rd, tmp, out_sc)
    return out.reshape(n_rays, _N1, 3)


def _tc_sample(pts_intervals, occ_values, rd, cam_loc, tmp):
    n_rays = pts_intervals.shape[0]
    out = pl.pallas_call(
        _body,
        grid=(n_rays // _R,),
        in_specs=[
            pl.BlockSpec((1, 3), lambda i: (0, 0)),
            pl.BlockSpec((_R, _N_STEPS), lambda i: (i, 0)),
            pl.BlockSpec((_R, _N_STEPS), lambda i: (i, 0)),
            pl.BlockSpec((_R, 3), lambda i: (i, 0)),
            pl.BlockSpec((_R, _N1), lambda i: (i, 0)),
        ],
        out_specs=pl.BlockSpec((_R, 3 * _N1), lambda i: (i, 0)),
        out_shape=jax.ShapeDtypeStruct((n_rays, 3 * _N1), jnp.float32),
    )(cam_loc, pts_intervals, occ_values, rd, tmp)
    return out
